# baseline (speedup 1.0000x reference)
import jax
import jax.numpy as jnp
from jax import lax
from jax.experimental import pallas as pl
from jax.experimental.pallas import tpu as pltpu
from jax.experimental.pallas import tpu_sc as plsc

NC = 2
NS = 16
NW = NC * NS
C = 128
NBUF = 4
LANES = 16


def _make_body(chunks, seq, d):
    nvec = d // LANES

    def body(x_hbm, w_hbm, pos_hbm, out_hbm, idx_v, pos_v,
             buf0, buf1, buf2, buf3,
             gsem0, gsem1, gsem2, gsem3,
             wsem0, wsem1, wsem2, wsem3):
        bufs = (buf0, buf1, buf2, buf3)
        gsems = (gsem0, gsem1, gsem2, gsem3)
        wsems = (wsem0, wsem1, wsem2, wsem3)

        wid = lax.axis_index("s") * NC + lax.axis_index("c")
        pltpu.sync_copy(x_hbm.at[wid], idx_v)
        pltpu.sync_copy(pos_hbm, pos_v)

        def gather_start(k, b):
            pltpu.make_async_copy(w_hbm.at[idx_v.at[k]], bufs[b], gsems[b]).start()

        def gather_wait(b):
            pltpu.make_async_copy(w_hbm.at[idx_v.at[0]], bufs[b], gsems[b]).wait()

        def wb_start(k, b):
            row_lo = pl.multiple_of(wid * (chunks * C) + k * C, C)
            pltpu.make_async_copy(bufs[b], out_hbm.at[pl.ds(row_lo, C)],
                                  wsems[b]).start()

        def wb_wait(b):
            pltpu.make_async_copy(bufs[b], out_hbm.at[pl.ds(0, C)],
                                  wsems[b]).wait()

        def add_pos(b, poff):
            rows = bufs[b]

            @pl.loop(0, C, unroll=2)
            def _(i):
                p = poff + i
                p = jnp.where(p >= seq, p - seq, p)
                for j in range(nvec):
                    sl = pl.ds(j * LANES, LANES)
                    plsc.addupdate(rows.at[i, sl], pos_v[p, sl])

        gather_start(0, 0)
        gather_start(1, 1)

        @pl.loop(0, chunks, step=NBUF)
        def _(g):
            for b in range(NBUF):
                k = g + b
                bn = (b + 2) % NBUF

                @pl.when(jnp.logical_and(k >= 2, k + 2 < chunks))
                def _():
                    wb_wait(bn)

                @pl.when(k + 2 < chunks)
                def _():
                    gather_start(k + 2, bn)

                gather_wait(b)
                add_pos(b, lax.rem(k * C, seq))
                wb_start(k, b)

        for b in range(NBUF):
            wb_wait(b)

    return body


def kernel(x, W, pos):
    B, S = x.shape
    V, d = W.shape
    n = B * S
    per_w = n // NW
    chunks = per_w // C
    assert n == NW * chunks * C and C <= S and d % LANES == 0

    x_r = x.reshape(NW, chunks, C).astype(jnp.int32)
    mesh = plsc.VectorSubcoreMesh(
        core_axis_name="c", subcore_axis_name="s",
        num_cores=NC, num_subcores=NS)
    run = pl.kernel(
        _make_body(chunks, S, d),
        out_type=jax.ShapeDtypeStruct((NW * chunks * C, d), jnp.float32),
        mesh=mesh,
        scratch_types=[
            pltpu.VMEM((chunks, C), jnp.int32),
            pltpu.VMEM((S, d), jnp.float32),
        ] + [pltpu.VMEM((C, d), jnp.float32)] * NBUF
          + [pltpu.SemaphoreType.DMA] * (2 * NBUF),
    )
    out = run(x_r, W, pos)
    return out.reshape(B, S, d)


# speedup vs baseline: 2.2671x; 2.2671x over previous
import jax
import jax.numpy as jnp
from jax import lax
from jax.experimental import pallas as pl
from jax.experimental.pallas import tpu as pltpu
from jax.experimental.pallas import tpu_sc as plsc

NC = 2
NS = 16
NW = NC * NS
NBUF = 3
LANES = 16
IXA = 96
IXB = 104


def _make_body(chunks, seq, d):
    nvec = d // LANES

    def body(x_hbm, w_hbm, pos_hbm, out_hbm, pos_v,
             buf0, buf1, buf2, ib0, ib1, ib2,
             gsem0, gsem1, gsem2, wsem0, wsem1, wsem2,
             isem0, isem1, isem2):
        bufs = (buf0, buf1, buf2)
        ibufs = (ib0, ib1, ib2)
        gsems = (gsem0, gsem1, gsem2)
        wsems = (wsem0, wsem1, wsem2)
        isems = (isem0, isem1, isem2)

        wid = lax.axis_index("s") * NC + lax.axis_index("c")
        base = wid * (chunks * seq)
        pltpu.sync_copy(pos_hbm, pos_v)

        def idx_start(k, j):
            pltpu.make_async_copy(x_hbm.at[pl.ds(base + k * seq, seq)],
                                  ibufs[j], isems[j]).start()

        def idx_wait(j):
            pltpu.make_async_copy(x_hbm.at[pl.ds(0, seq)],
                                  ibufs[j], isems[j]).wait()

        def gather_descs(b, j):
            return (
                pltpu.make_async_copy(w_hbm.at[ibufs[j].at[pl.ds(0, IXA)]],
                                      bufs[b].at[pl.ds(0, IXA)], gsems[b]),
                pltpu.make_async_copy(w_hbm.at[ibufs[j].at[pl.ds(IXA, IXB)]],
                                      bufs[b].at[pl.ds(IXA, IXB)], gsems[b]),
            )

        def gather_start(b, j):
            for dsc in gather_descs(b, j):
                dsc.start()

        def gather_wait(b):
            for dsc in gather_descs(b, 0):
                dsc.wait()

        def wb_start(k, b):
            pltpu.make_async_copy(bufs[b], out_hbm.at[wid, k], wsems[b]).start()

        def wb_wait(b):
            pltpu.make_async_copy(bufs[b], out_hbm.at[wid, 0], wsems[b]).wait()

        def add_pos(b):
            rows = bufs[b]

            @pl.loop(0, seq, unroll=2)
            def _(i):
                for j in range(nvec):
                    sl = pl.ds(j * LANES, LANES)
                    plsc.addupdate(rows.at[i, sl], pos_v[i, sl])

        idx_start(0, 0)
        idx_start(1, 1)
        idx_wait(0)
        gather_start(0, 0)

        main = chunks - (chunks % NBUF and 2)

        @pl.loop(0, main, step=NBUF)
        def _(g):
            for b in range(NBUF):
                k = g + b
                bn = (b + 1) % NBUF
                bn2 = (b + 2) % NBUF

                @pl.when(k + 2 < chunks)
                def _():
                    idx_start(k + 2, bn2)

                @pl.when(k >= 2)
                def _():
                    wb_wait(bn)

                @pl.when(k + 1 < chunks)
                def _():
                    idx_wait(bn)
                    gather_start(bn, bn)

                gather_wait(b)
                add_pos(b)
                wb_start(k, b)

        for k in range(main, chunks):
            b, bn = k % NBUF, (k + 1) % NBUF
            if k + 2 < chunks:
                idx_start(k + 2, (k + 2) % NBUF)
            if k >= 2:
                wb_wait(bn)
            if k + 1 < chunks:
                idx_wait(bn)
                gather_start(bn, bn)
            gather_wait(b)
            add_pos(b)
            wb_start(k, b)

        for k in range(chunks - 2, chunks):
            wb_wait(k % NBUF)

    return body


def kernel(x, W, pos):
    B, S = x.shape
    V, d = W.shape
    n = B * S
    per_w = n // NW
    chunks = per_w // S
    assert n == NW * chunks * S and S == IXA + IXB and d % LANES == 0

    x_r = x.reshape(n).astype(jnp.int32)
    mesh = plsc.VectorSubcoreMesh(
        core_axis_name="c", subcore_axis_name="s",
        num_cores=NC, num_subcores=NS)
    run = pl.kernel(
        _make_body(chunks, S, d),
        out_type=jax.ShapeDtypeStruct((NW, chunks, S, d), jnp.float32),
        mesh=mesh,
        scratch_types=[
            pltpu.VMEM((S, d), jnp.float32),
        ] + [pltpu.VMEM((S, d), jnp.float32)] * NBUF
          + [pltpu.VMEM((S,), jnp.int32)] * NBUF
          + [pltpu.SemaphoreType.DMA] * (3 * NBUF),
    )
    out = run(x_r, W, pos)
    return out.reshape(B, S, d)
